# baseline (speedup 1.0000x reference)
import functools

import jax
import jax.numpy as jnp
from jax import lax
from jax.experimental import pallas as pl
from jax.experimental.pallas import tpu as pltpu
from jax.experimental.pallas import tpu_sc as plsc

NC = 2
NS = 16
NW = NC * NS
LANES = 16


def _node_padding(N):
    RPT = (-(-N // NS) + 7) // 8 * 8
    while RPT % 5:
        RPT += 8
    return RPT, RPT * NS


def _sc_count(edge_index, N):
    E = edge_index.shape[1]
    EW = E // NW
    C = 80
    NB = 5
    n_waves = EW // (C * NB)
    RPT, NP = _node_padding(N)

    mesh = plsc.VectorSubcoreMesh(core_axis_name="c", subcore_axis_name="s", num_cores=NC, num_subcores=NS)

    @functools.partial(
        pl.kernel,
        out_type=jax.ShapeDtypeStruct((NC, NP, LANES), jnp.float32),
        mesh=mesh,
        scratch_types=[
            pltpu.VMEM((NB, C), jnp.int32),
            pltpu.VMEM((C, LANES), jnp.float32),
            pltpu.VMEM((RPT, LANES), jnp.float32),
            pltpu.VMEM_SHARED((NP, LANES), jnp.float32),
            pltpu.SemaphoreType.DMA,
            pltpu.SemaphoreType.DMA,
        ],
        compiler_params=pltpu.CompilerParams(use_tc_tiling_on_sc=False))
    def k(ei_hbm, cnt_hbm, idx_v, ones_v, zcnt_v, cnt_sp, si, ss):
        cid = lax.axis_index("c")
        sid = lax.axis_index("s")
        wid = cid * NS + sid

        def orow(i, _):
            ones_v[i, pl.ds(0, LANES)] = jnp.ones((LANES,), jnp.float32)
            return 0
        lax.fori_loop(0, C, orow, 0)

        def zcrow(i, _):
            zcnt_v[i, pl.ds(0, LANES)] = jnp.zeros((LANES,), jnp.float32)
            return 0
        lax.fori_loop(0, RPT, zcrow, 0)
        pltpu.sync_copy(zcnt_v, cnt_sp.at[pl.ds(sid * RPT, RPT)])
        plsc.subcore_barrier()

        def wave(w, _):
            base0 = wid * EW + w * (C * NB)
            loads = [pltpu.async_copy(
                ei_hbm.at[1, pl.ds(base0 + b * C, C)], idx_v.at[b], si)
                for b in range(NB)]
            for d in loads:
                d.wait()
            scatters = [pltpu.async_copy(
                ones_v, cnt_sp.at[idx_v.at[b]], ss, add=True)
                for b in range(NB)]
            for d in scatters:
                d.wait()
            return 0
        lax.fori_loop(0, n_waves, wave, 0)

        plsc.subcore_barrier()
        pltpu.sync_copy(cnt_sp.at[pl.ds(sid * RPT, RPT)],
                        cnt_hbm.at[cid, pl.ds(sid * RPT, RPT)])

    return k(edge_index)


def _sc_aggregate(x, edge_index):
    N, D = x.shape
    E = edge_index.shape[1]
    EW = E // NW
    C = 40
    NB = 5
    n_waves = EW // (C * NB)
    RPT, NP = _node_padding(N)
    ZR = RPT // 10

    scratch = [
        pltpu.VMEM((NB, 2, C), jnp.int32),
        pltpu.VMEM((NB, C, D), jnp.float32),
        pltpu.VMEM((ZR, D), jnp.float32),
        pltpu.VMEM_SHARED((NP, D), jnp.float32),
        pltpu.SemaphoreType.DMA,
        pltpu.SemaphoreType.DMA,
        pltpu.SemaphoreType.DMA,
    ]

    mesh = plsc.VectorSubcoreMesh(core_axis_name="c", subcore_axis_name="s", num_cores=NC, num_subcores=NS)

    def body(x_hbm, ei_hbm, agg_hbm, idx_v, rows_v, zero_v, agg_sp,
             si, sg, ss):
        cid = lax.axis_index("c")
        sid = lax.axis_index("s")
        wid = cid * NS + sid

        def zrow(i, _):
            for k in range(D // LANES):
                zero_v[i, pl.ds(LANES * k, LANES)] = jnp.zeros(
                    (LANES,), jnp.float32)
            return 0
        lax.fori_loop(0, ZR, zrow, 0)
        for j in range(RPT // ZR):
            pltpu.sync_copy(zero_v, agg_sp.at[pl.ds(sid * RPT + j * ZR, ZR)])
        plsc.subcore_barrier()

        def wave(w, _):
            base0 = wid * EW + w * (C * NB)
            loads = []
            for b in range(NB):
                base = base0 + b * C
                loads.append(pltpu.async_copy(
                    ei_hbm.at[0, pl.ds(base, C)], idx_v.at[b, 0], si))
                loads.append(pltpu.async_copy(
                    ei_hbm.at[1, pl.ds(base, C)], idx_v.at[b, 1], si))
            for d in loads:
                d.wait()
            gathers = [pltpu.async_copy(x_hbm.at[idx_v.at[b, 0]],
                                        rows_v.at[b], sg)
                       for b in range(NB)]
            for d in gathers:
                d.wait()
            scatters = [pltpu.async_copy(
                rows_v.at[b], agg_sp.at[idx_v.at[b, 1]], ss, add=True)
                for b in range(NB)]
            for d in scatters:
                d.wait()
            return 0
        lax.fori_loop(0, n_waves, wave, 0)

        plsc.subcore_barrier()
        pltpu.sync_copy(agg_sp.at[pl.ds(sid * RPT, RPT)],
                        agg_hbm.at[cid, pl.ds(sid * RPT, RPT)])

    f = pl.kernel(body,
                  out_type=jax.ShapeDtypeStruct((NC, NP, D), jnp.float32),
                  mesh=mesh, scratch_types=scratch,
                  compiler_params=pltpu.CompilerParams(
                      use_tc_tiling_on_sc=False))
    return f(x, edge_index)


def _tc_node_update(aggp, cntp, x, wl_t, bl, wr_t, out_dtype=jnp.float32):
    N, D = x.shape
    BN = 2000
    grid = (N // BN,)

    def body(agg_ref, cnt_ref, x_ref, wl_ref, bl_ref, wr_ref, out_ref):
        agg = agg_ref[0] + agg_ref[1]
        cnt = cnt_ref[0, :, 0:1] + cnt_ref[1, :, 0:1]
        mean = agg / jnp.maximum(cnt, 1.0)
        h = jnp.dot(mean, wl_ref[...], preferred_element_type=jnp.float32)
        h = h + jnp.dot(x_ref[...], wr_ref[...],
                        preferred_element_type=jnp.float32)
        out_ref[...] = jnp.maximum(h + bl_ref[...], 0.0).astype(out_dtype)

    return pl.pallas_call(
        body,
        grid=grid,
        in_specs=[
            pl.BlockSpec((NC, BN, D), lambda i: (0, i, 0)),
            pl.BlockSpec((NC, BN, LANES), lambda i: (0, i, 0)),
            pl.BlockSpec((BN, D), lambda i: (i, 0)),
            pl.BlockSpec((D, D), lambda i: (0, 0)),
            pl.BlockSpec((1, D), lambda i: (0, 0)),
            pl.BlockSpec((D, D), lambda i: (0, 0)),
        ],
        out_specs=pl.BlockSpec((BN, D), lambda i: (i, 0)),
        out_shape=jax.ShapeDtypeStruct((N, D), out_dtype),
    )(aggp, cntp, x, wl_t, bl, wr_t)


def _sc_gather_pairs(h, edge_index):
    N, D = h.shape
    E = edge_index.shape[1]
    EW = E // NW
    C = 80
    NB = 5
    n_waves = EW // (C * NB)

    dt = h.dtype
    mesh = plsc.VectorSubcoreMesh(core_axis_name="c", subcore_axis_name="s", num_cores=NC, num_subcores=NS)

    @functools.partial(
        pl.kernel,
        out_type=(jax.ShapeDtypeStruct((E, D), dt),
                  jax.ShapeDtypeStruct((E, D), dt)),
        mesh=mesh,
        scratch_types=[
            pltpu.VMEM((NB, 2, C), jnp.int32),
            pltpu.VMEM((NB, C, D), dt),
            pltpu.VMEM((NB, C, D), dt),
            pltpu.SemaphoreType.DMA,
            pltpu.SemaphoreType.DMA,
            pltpu.SemaphoreType.DMA,
        ],
        compiler_params=pltpu.CompilerParams(use_tc_tiling_on_sc=False))
    def k(h_hbm, ei_hbm, gv_hbm, gu_hbm, idx_v, rv, ru, si, sg, so):
        cid = lax.axis_index("c")
        sid = lax.axis_index("s")
        wid = cid * NS + sid

        def wave(w, _):
            base0 = wid * EW + w * (C * NB)
            loads = []
            for b in range(NB):
                base = base0 + b * C
                loads.append(pltpu.async_copy(
                    ei_hbm.at[0, pl.ds(base, C)], idx_v.at[b, 0], si))
                loads.append(pltpu.async_copy(
                    ei_hbm.at[1, pl.ds(base, C)], idx_v.at[b, 1], si))
            for d in loads:
                d.wait()
            gathers = []
            for b in range(NB):
                gathers.append(pltpu.async_copy(
                    h_hbm.at[idx_v.at[b, 0]], rv.at[b], sg))
                gathers.append(pltpu.async_copy(
                    h_hbm.at[idx_v.at[b, 1]], ru.at[b], sg))
            for d in gathers:
                d.wait()
            stores = []
            for b in range(NB):
                base = base0 + b * C
                stores.append(pltpu.async_copy(
                    rv.at[b], gv_hbm.at[pl.ds(base, C)], so))
                stores.append(pltpu.async_copy(
                    ru.at[b], gu_hbm.at[pl.ds(base, C)], so))
            for d in stores:
                d.wait()
            return 0
        lax.fori_loop(0, n_waves, wave, 0)

    return k(h, edge_index)


def _tc_edge_mlp(gv, gu, ef, w1v_t, w1u_t, w1e_t, b1,
                 w2_t, b2, w3_t, b3, w4_t, b4):
    E, D = gv.shape
    DE = ef.shape[1]
    H1 = w1v_t.shape[1]
    H2 = w2_t.shape[1]
    H3 = w3_t.shape[1]
    OUT = w4_t.shape[1]
    BE = 2000
    grid = (E // BE,)

    bf = jnp.bfloat16

    def body(gv_ref, gu_ref, ef_ref, w1v_ref, w1u_ref, w1e_ref, b1_ref,
             w2_ref, b2_ref, w3_ref, b3_ref, w4_ref, b4_ref, out_ref):
        h = jnp.dot(gv_ref[...], w1v_ref[...],
                    preferred_element_type=jnp.float32)
        h = h + jnp.dot(gu_ref[...], w1u_ref[...],
                        preferred_element_type=jnp.float32)
        h = h + jnp.dot(ef_ref[...], w1e_ref[...],
                        preferred_element_type=jnp.float32)
        h = jnp.maximum(h + b1_ref[...], 0.0).astype(bf)
        h = jnp.maximum(jnp.dot(h, w2_ref[...],
                                preferred_element_type=jnp.float32)
                        + b2_ref[...], 0.0).astype(bf)
        h = jnp.maximum(jnp.dot(h, w3_ref[...],
                                preferred_element_type=jnp.float32)
                        + b3_ref[...], 0.0).astype(bf)
        out_ref[...] = jnp.dot(h, w4_ref[...],
                               preferred_element_type=jnp.float32) + b4_ref[...]

    return pl.pallas_call(
        body,
        grid=grid,
        in_specs=[
            pl.BlockSpec((BE, D), lambda i: (i, 0)),
            pl.BlockSpec((BE, D), lambda i: (i, 0)),
            pl.BlockSpec((BE, DE), lambda i: (i, 0)),
            pl.BlockSpec((D, H1), lambda i: (0, 0)),
            pl.BlockSpec((D, H1), lambda i: (0, 0)),
            pl.BlockSpec((DE, H1), lambda i: (0, 0)),
            pl.BlockSpec((1, H1), lambda i: (0, 0)),
            pl.BlockSpec((H1, H2), lambda i: (0, 0)),
            pl.BlockSpec((1, H2), lambda i: (0, 0)),
            pl.BlockSpec((H2, H3), lambda i: (0, 0)),
            pl.BlockSpec((1, H3), lambda i: (0, 0)),
            pl.BlockSpec((H3, OUT), lambda i: (0, 0)),
            pl.BlockSpec((1, OUT), lambda i: (0, 0)),
        ],
        out_specs=pl.BlockSpec((BE, OUT), lambda i: (i, 0)),
        out_shape=jax.ShapeDtypeStruct((E, OUT), jnp.float32),
    )(gv, gu, ef, w1v_t, w1u_t, w1e_t, b1, w2_t, b2, w3_t, b3, w4_t, b4)


def kernel(x, edge_index, edge_features, num_nodes,
           W_l, b_l, W_r, W1, b1, W2, b2, W3, b3, W4, b4):
    del num_nodes
    D = x.shape[1]

    wl_t = W_l.T
    wr_t = W_r.T
    bl = b_l.reshape(1, -1)

    cntp = _sc_count(edge_index, x.shape[0])
    agg1 = _sc_aggregate(x, edge_index)
    h1 = _tc_node_update(agg1, cntp, x, wl_t, bl, wr_t)
    agg2 = _sc_aggregate(h1, edge_index)
    h2 = _tc_node_update(agg2, cntp, h1, wl_t, bl, wr_t,
                         out_dtype=jnp.bfloat16)

    gv, gu = _sc_gather_pairs(h2, edge_index)
    bf = jnp.bfloat16
    pred = _tc_edge_mlp(
        gv, gu, edge_features.astype(bf),
        W1[:, :D].T.astype(bf), W1[:, D:2 * D].T.astype(bf),
        W1[:, 2 * D:].T.astype(bf), b1.reshape(1, -1),
        W2.T.astype(bf), b2.reshape(1, -1), W3.T.astype(bf),
        b3.reshape(1, -1), W4.T.astype(bf), b4.reshape(1, -1))
    return pred


# speedup vs baseline: 1.4242x; 1.4242x over previous
import functools

import jax
import jax.numpy as jnp
from jax import lax
from jax.experimental import pallas as pl
from jax.experimental.pallas import tpu as pltpu
from jax.experimental.pallas import tpu_sc as plsc

NC = 2
NS = 16
NW = NC * NS
LANES = 16


def _node_padding(N):
    RPT = (-(-N // NS) + 7) // 8 * 8
    while RPT % 5:
        RPT += 8
    return RPT, RPT * NS


def _sc_count(edge_index, N):
    E = edge_index.shape[1]
    EW = E // NW
    C = 80
    NB = 5
    n_waves = EW // (C * NB)
    RPT, NP = _node_padding(N)

    mesh = plsc.VectorSubcoreMesh(core_axis_name="c", subcore_axis_name="s", num_cores=NC, num_subcores=NS)

    @functools.partial(
        pl.kernel,
        out_type=jax.ShapeDtypeStruct((NC, NP, LANES), jnp.float32),
        mesh=mesh,
        scratch_types=[
            pltpu.VMEM((NB, C), jnp.int32),
            pltpu.VMEM((C, LANES), jnp.float32),
            pltpu.VMEM((RPT, LANES), jnp.float32),
            pltpu.VMEM_SHARED((NP, LANES), jnp.float32),
            pltpu.SemaphoreType.DMA,
            pltpu.SemaphoreType.DMA,
        ],
        compiler_params=pltpu.CompilerParams(use_tc_tiling_on_sc=False))
    def k(ei_hbm, cnt_hbm, idx_v, ones_v, zcnt_v, cnt_sp, si, ss):
        cid = lax.axis_index("c")
        sid = lax.axis_index("s")
        wid = cid * NS + sid

        def orow(i, _):
            ones_v[i, pl.ds(0, LANES)] = jnp.ones((LANES,), jnp.float32)
            return 0
        lax.fori_loop(0, C, orow, 0)

        def zcrow(i, _):
            zcnt_v[i, pl.ds(0, LANES)] = jnp.zeros((LANES,), jnp.float32)
            return 0
        lax.fori_loop(0, RPT, zcrow, 0)
        pltpu.sync_copy(zcnt_v, cnt_sp.at[pl.ds(sid * RPT, RPT)])
        plsc.subcore_barrier()

        def wave(w, _):
            base0 = wid * EW + w * (C * NB)
            loads = [pltpu.async_copy(
                ei_hbm.at[1, pl.ds(base0 + b * C, C)], idx_v.at[b], si)
                for b in range(NB)]
            for d in loads:
                d.wait()
            scatters = [pltpu.async_copy(
                ones_v, cnt_sp.at[idx_v.at[b]], ss, add=True)
                for b in range(NB)]
            for d in scatters:
                d.wait()
            return 0
        lax.fori_loop(0, n_waves, wave, 0)

        plsc.subcore_barrier()
        pltpu.sync_copy(cnt_sp.at[pl.ds(sid * RPT, RPT)],
                        cnt_hbm.at[cid, pl.ds(sid * RPT, RPT)])

    return k(edge_index)


def _sc_aggregate(x, edge_index):
    N, D = x.shape
    E = edge_index.shape[1]
    EW = E // NW
    C = 40
    NB = 5
    n_waves = EW // (C * NB)
    RPT, NP = _node_padding(N)
    ZR = RPT // 10

    scratch = [
        pltpu.VMEM((NB, 2, C), jnp.int32),
        pltpu.VMEM((NB, C, D), jnp.float32),
        pltpu.VMEM((ZR, D), jnp.float32),
        pltpu.VMEM_SHARED((NP, D), jnp.float32),
        pltpu.SemaphoreType.DMA,
        pltpu.SemaphoreType.DMA,
        pltpu.SemaphoreType.DMA,
    ]

    mesh = plsc.VectorSubcoreMesh(core_axis_name="c", subcore_axis_name="s", num_cores=NC, num_subcores=NS)

    def body(x_hbm, ei_hbm, agg_hbm, idx_v, rows_v, zero_v, agg_sp,
             si, sg, ss):
        cid = lax.axis_index("c")
        sid = lax.axis_index("s")
        wid = cid * NS + sid

        def zrow(i, _):
            for k in range(D // LANES):
                zero_v[i, pl.ds(LANES * k, LANES)] = jnp.zeros(
                    (LANES,), jnp.float32)
            return 0
        lax.fori_loop(0, ZR, zrow, 0)
        for j in range(RPT // ZR):
            pltpu.sync_copy(zero_v, agg_sp.at[pl.ds(sid * RPT + j * ZR, ZR)])
        plsc.subcore_barrier()

        def wave(w, _):
            base0 = wid * EW + w * (C * NB)
            loads = []
            for b in range(NB):
                base = base0 + b * C
                loads.append(pltpu.async_copy(
                    ei_hbm.at[0, pl.ds(base, C)], idx_v.at[b, 0], si))
                loads.append(pltpu.async_copy(
                    ei_hbm.at[1, pl.ds(base, C)], idx_v.at[b, 1], si))
            for d in loads:
                d.wait()
            gathers = [pltpu.async_copy(x_hbm.at[idx_v.at[b, 0]],
                                        rows_v.at[b], sg)
                       for b in range(NB)]
            for d in gathers:
                d.wait()
            scatters = [pltpu.async_copy(
                rows_v.at[b], agg_sp.at[idx_v.at[b, 1]], ss, add=True)
                for b in range(NB)]
            for d in scatters:
                d.wait()
            return 0
        lax.fori_loop(0, n_waves, wave, 0)

        plsc.subcore_barrier()
        pltpu.sync_copy(agg_sp.at[pl.ds(sid * RPT, RPT)],
                        agg_hbm.at[cid, pl.ds(sid * RPT, RPT)])

    f = pl.kernel(body,
                  out_type=jax.ShapeDtypeStruct((NC, NP, D), jnp.float32),
                  mesh=mesh, scratch_types=scratch,
                  compiler_params=pltpu.CompilerParams(
                      use_tc_tiling_on_sc=False))
    return f(x, edge_index)


def _tc_node_update(aggp, cntp, x, wl_t, bl, wr_t, out_dtype=jnp.float32):
    N, D = x.shape
    BN = 2000
    grid = (N // BN,)

    def body(agg_ref, cnt_ref, x_ref, wl_ref, bl_ref, wr_ref, out_ref):
        agg = agg_ref[0] + agg_ref[1]
        cnt = cnt_ref[0, :, 0:1] + cnt_ref[1, :, 0:1]
        mean = agg / jnp.maximum(cnt, 1.0)
        h = jnp.dot(mean, wl_ref[...], preferred_element_type=jnp.float32)
        h = h + jnp.dot(x_ref[...], wr_ref[...],
                        preferred_element_type=jnp.float32)
        out_ref[...] = jnp.maximum(h + bl_ref[...], 0.0).astype(out_dtype)

    return pl.pallas_call(
        body,
        grid=grid,
        in_specs=[
            pl.BlockSpec((NC, BN, D), lambda i: (0, i, 0)),
            pl.BlockSpec((NC, BN, LANES), lambda i: (0, i, 0)),
            pl.BlockSpec((BN, D), lambda i: (i, 0)),
            pl.BlockSpec((D, D), lambda i: (0, 0)),
            pl.BlockSpec((1, D), lambda i: (0, 0)),
            pl.BlockSpec((D, D), lambda i: (0, 0)),
        ],
        out_specs=pl.BlockSpec((BN, D), lambda i: (i, 0)),
        out_shape=jax.ShapeDtypeStruct((N, D), out_dtype),
    )(aggp, cntp, x, wl_t, bl, wr_t)


def _sc_gather_pairs(h, edge_index):
    N, D = h.shape
    E = edge_index.shape[1]
    EW = E // NW
    C = 80
    NB = 5
    n_waves = EW // (C * NB)

    dt = h.dtype
    mesh = plsc.VectorSubcoreMesh(core_axis_name="c", subcore_axis_name="s", num_cores=NC, num_subcores=NS)

    @functools.partial(
        pl.kernel,
        out_type=(jax.ShapeDtypeStruct((E, D), dt),
                  jax.ShapeDtypeStruct((E, D), dt)),
        mesh=mesh,
        scratch_types=[
            pltpu.VMEM((NB, 2, C), jnp.int32),
            pltpu.VMEM((NB, C, D), dt),
            pltpu.VMEM((NB, C, D), dt),
            pltpu.SemaphoreType.DMA,
            pltpu.SemaphoreType.DMA,
            pltpu.SemaphoreType.DMA,
        ],
        compiler_params=pltpu.CompilerParams(use_tc_tiling_on_sc=False))
    def k(h_hbm, ei_hbm, gv_hbm, gu_hbm, idx_v, rv, ru, si, sg, so):
        cid = lax.axis_index("c")
        sid = lax.axis_index("s")
        wid = cid * NS + sid

        def wave(w, _):
            base0 = wid * EW + w * (C * NB)
            loads = []
            for b in range(NB):
                base = base0 + b * C
                loads.append(pltpu.async_copy(
                    ei_hbm.at[0, pl.ds(base, C)], idx_v.at[b, 0], si))
                loads.append(pltpu.async_copy(
                    ei_hbm.at[1, pl.ds(base, C)], idx_v.at[b, 1], si))
            for d in loads:
                d.wait()
            gathers = []
            for b in range(NB):
                gathers.append(pltpu.async_copy(
                    h_hbm.at[idx_v.at[b, 0]], rv.at[b], sg))
                gathers.append(pltpu.async_copy(
                    h_hbm.at[idx_v.at[b, 1]], ru.at[b], sg))
            for d in gathers:
                d.wait()
            stores = []
            for b in range(NB):
                base = base0 + b * C
                stores.append(pltpu.async_copy(
                    rv.at[b], gv_hbm.at[pl.ds(base, C)], so))
                stores.append(pltpu.async_copy(
                    ru.at[b], gu_hbm.at[pl.ds(base, C)], so))
            for d in stores:
                d.wait()
            return 0
        lax.fori_loop(0, n_waves, wave, 0)

    return k(h, edge_index)


def _tc_edge_mlp(gv, gu, ef, w1v_t, w1u_t, w1e_t, b1,
                 w2_t, b2, w3_t, b3, w4_t, b4):
    E, D = gv.shape
    DE = ef.shape[1]
    H1 = w1v_t.shape[1]
    H2 = w2_t.shape[1]
    H3 = w3_t.shape[1]
    OUT = w4_t.shape[1]
    BE = 2000
    grid = (E // BE,)

    bf = jnp.bfloat16

    def body(gv_ref, gu_ref, ef_ref, w1v_ref, w1u_ref, w1e_ref, b1_ref,
             w2_ref, b2_ref, w3_ref, b3_ref, w4_ref, b4_ref, out_ref):
        h = jnp.dot(gv_ref[...].astype(bf), w1v_ref[...],
                    preferred_element_type=jnp.float32)
        h = h + jnp.dot(gu_ref[...].astype(bf), w1u_ref[...],
                        preferred_element_type=jnp.float32)
        h = h + jnp.dot(ef_ref[...].astype(bf), w1e_ref[...],
                        preferred_element_type=jnp.float32)
        h = jnp.maximum(h + b1_ref[...], 0.0).astype(bf)
        h = jnp.maximum(jnp.dot(h, w2_ref[...],
                                preferred_element_type=jnp.float32)
                        + b2_ref[...], 0.0).astype(bf)
        h = jnp.maximum(jnp.dot(h, w3_ref[...],
                                preferred_element_type=jnp.float32)
                        + b3_ref[...], 0.0).astype(bf)
        out_ref[...] = jnp.dot(h, w4_ref[...],
                               preferred_element_type=jnp.float32) + b4_ref[...]

    return pl.pallas_call(
        body,
        grid=grid,
        in_specs=[
            pl.BlockSpec((BE, D), lambda i: (i, 0)),
            pl.BlockSpec((BE, D), lambda i: (i, 0)),
            pl.BlockSpec((BE, DE), lambda i: (i, 0)),
            pl.BlockSpec((D, H1), lambda i: (0, 0)),
            pl.BlockSpec((D, H1), lambda i: (0, 0)),
            pl.BlockSpec((DE, H1), lambda i: (0, 0)),
            pl.BlockSpec((1, H1), lambda i: (0, 0)),
            pl.BlockSpec((H1, H2), lambda i: (0, 0)),
            pl.BlockSpec((1, H2), lambda i: (0, 0)),
            pl.BlockSpec((H2, H3), lambda i: (0, 0)),
            pl.BlockSpec((1, H3), lambda i: (0, 0)),
            pl.BlockSpec((H3, OUT), lambda i: (0, 0)),
            pl.BlockSpec((1, OUT), lambda i: (0, 0)),
        ],
        out_specs=pl.BlockSpec((BE, OUT), lambda i: (i, 0)),
        out_shape=jax.ShapeDtypeStruct((E, OUT), jnp.float32),
    )(gv, gu, ef, w1v_t, w1u_t, w1e_t, b1, w2_t, b2, w3_t, b3, w4_t, b4)


def kernel(x, edge_index, edge_features, num_nodes,
           W_l, b_l, W_r, W1, b1, W2, b2, W3, b3, W4, b4):
    del num_nodes
    D = x.shape[1]

    wl_t = W_l.T
    wr_t = W_r.T
    bl = b_l.reshape(1, -1)

    cntp = _sc_count(edge_index, x.shape[0])
    agg1 = _sc_aggregate(x, edge_index)
    h1 = _tc_node_update(agg1, cntp, x, wl_t, bl, wr_t)
    agg2 = _sc_aggregate(h1, edge_index)
    h2 = _tc_node_update(agg2, cntp, h1, wl_t, bl, wr_t)

    gv, gu = _sc_gather_pairs(h2, edge_index)
    bf = jnp.bfloat16
    pred = _tc_edge_mlp(
        gv, gu, edge_features,
        W1[:, :D].T.astype(bf), W1[:, D:2 * D].T.astype(bf),
        W1[:, 2 * D:].T.astype(bf), b1.reshape(1, -1),
        W2.T.astype(bf), b2.reshape(1, -1), W3.T.astype(bf),
        b3.reshape(1, -1), W4.T.astype(bf), b4.reshape(1, -1))
    return pred


# speedup vs baseline: 1.6001x; 1.1235x over previous
import functools

import jax
import jax.numpy as jnp
from jax import lax
from jax.experimental import pallas as pl
from jax.experimental.pallas import tpu as pltpu
from jax.experimental.pallas import tpu_sc as plsc

NC = 2
NS = 16
NW = NC * NS
LANES = 16


def _node_padding(N):
    RPT = (-(-N // NS) + 7) // 8 * 8
    while RPT % 5:
        RPT += 8
    return RPT, RPT * NS


def _sc_aggregate(x, edge_index, with_count=False):
    N, D = x.shape
    E = edge_index.shape[1]
    EW = E // NW
    C = 40
    NB = 5
    n_waves = EW // (C * NB)
    RPT, NP = _node_padding(N)
    ZR = RPT // 10

    out_type = [jax.ShapeDtypeStruct((NC, NP, D), jnp.float32)]
    scratch = [
        pltpu.VMEM((NB, 2, C), jnp.int32),
        pltpu.VMEM((NB, C, D), jnp.float32),
        pltpu.VMEM((ZR, D), jnp.float32),
        pltpu.VMEM_SHARED((NP, D), jnp.float32),
        pltpu.SemaphoreType.DMA,
        pltpu.SemaphoreType.DMA,
        pltpu.SemaphoreType.DMA,
    ]
    if with_count:
        out_type.append(jax.ShapeDtypeStruct((NC, NP, LANES), jnp.float32))
        scratch += [
            pltpu.VMEM((C, LANES), jnp.float32),
            pltpu.VMEM((ZR, LANES), jnp.float32),
            pltpu.VMEM_SHARED((NP, LANES), jnp.float32),
        ]

    mesh = plsc.VectorSubcoreMesh(core_axis_name="c", subcore_axis_name="s", num_cores=NC, num_subcores=NS)

    def body(x_hbm, ei_hbm, *refs):
        if with_count:
            (agg_hbm, cnt_hbm, idx_v, rows_v, zero_v, agg_sp, si, sg, ss,
             ones_v, zcnt_v, cnt_sp) = refs
        else:
            (agg_hbm, idx_v, rows_v, zero_v, agg_sp, si, sg, ss) = refs
        cid = lax.axis_index("c")
        sid = lax.axis_index("s")
        wid = cid * NS + sid

        def zrow(i, _):
            for k in range(D // LANES):
                zero_v[i, pl.ds(LANES * k, LANES)] = jnp.zeros(
                    (LANES,), jnp.float32)
            return 0
        lax.fori_loop(0, ZR, zrow, 0)
        for j in range(RPT // ZR):
            pltpu.sync_copy(zero_v, agg_sp.at[pl.ds(sid * RPT + j * ZR, ZR)])
        if with_count:
            def orow(i, _):
                ones_v[i, pl.ds(0, LANES)] = jnp.ones((LANES,), jnp.float32)
                return 0
            lax.fori_loop(0, C, orow, 0)

            def zcrow(i, _):
                zcnt_v[i, pl.ds(0, LANES)] = jnp.zeros((LANES,), jnp.float32)
                return 0
            lax.fori_loop(0, ZR, zcrow, 0)
            for j in range(RPT // ZR):
                pltpu.sync_copy(
                    zcnt_v, cnt_sp.at[pl.ds(sid * RPT + j * ZR, ZR)])
        plsc.subcore_barrier()

        def wave(w, _):
            base0 = wid * EW + w * (C * NB)
            loads = []
            for b in range(NB):
                base = base0 + b * C
                loads.append(pltpu.async_copy(
                    ei_hbm.at[0, pl.ds(base, C)], idx_v.at[b, 0], si))
                loads.append(pltpu.async_copy(
                    ei_hbm.at[1, pl.ds(base, C)], idx_v.at[b, 1], si))
            for d in loads:
                d.wait()
            gathers = [pltpu.async_copy(x_hbm.at[idx_v.at[b, 0]],
                                        rows_v.at[b], sg)
                       for b in range(NB)]
            for d in gathers:
                d.wait()
            scatters = []
            for b in range(NB):
                scatters.append(pltpu.async_copy(
                    rows_v.at[b], agg_sp.at[idx_v.at[b, 1]], ss, add=True))
                if with_count:
                    scatters.append(pltpu.async_copy(
                        ones_v, cnt_sp.at[idx_v.at[b, 1]], ss, add=True))
            for d in scatters:
                d.wait()
            return 0
        lax.fori_loop(0, n_waves, wave, 0)

        plsc.subcore_barrier()
        pltpu.sync_copy(agg_sp.at[pl.ds(sid * RPT, RPT)],
                        agg_hbm.at[cid, pl.ds(sid * RPT, RPT)])
        if with_count:
            pltpu.sync_copy(cnt_sp.at[pl.ds(sid * RPT, RPT)],
                            cnt_hbm.at[cid, pl.ds(sid * RPT, RPT)])

    f = pl.kernel(body, out_type=tuple(out_type), mesh=mesh,
                  scratch_types=scratch,
                  compiler_params=pltpu.CompilerParams(
                      use_tc_tiling_on_sc=False))
    out = f(x, edge_index)
    return out if with_count else out[0]


def _tc_node_update(aggp, cntp, x, wl_t, bl, wr_t, out_dtype=jnp.float32):
    N, D = x.shape
    BN = 2000
    grid = (N // BN,)

    def body(agg_ref, cnt_ref, x_ref, wl_ref, bl_ref, wr_ref, out_ref):
        agg = agg_ref[0] + agg_ref[1]
        cnt = cnt_ref[0, :, 0:1] + cnt_ref[1, :, 0:1]
        mean = agg / jnp.maximum(cnt, 1.0)
        h = jnp.dot(mean, wl_ref[...], preferred_element_type=jnp.float32)
        h = h + jnp.dot(x_ref[...], wr_ref[...],
                        preferred_element_type=jnp.float32)
        out_ref[...] = jnp.maximum(h + bl_ref[...], 0.0).astype(out_dtype)

    return pl.pallas_call(
        body,
        grid=grid,
        in_specs=[
            pl.BlockSpec((NC, BN, D), lambda i: (0, i, 0)),
            pl.BlockSpec((NC, BN, LANES), lambda i: (0, i, 0)),
            pl.BlockSpec((BN, D), lambda i: (i, 0)),
            pl.BlockSpec((D, D), lambda i: (0, 0)),
            pl.BlockSpec((1, D), lambda i: (0, 0)),
            pl.BlockSpec((D, D), lambda i: (0, 0)),
        ],
        out_specs=pl.BlockSpec((BN, D), lambda i: (i, 0)),
        out_shape=jax.ShapeDtypeStruct((N, D), out_dtype),
    )(aggp, cntp, x, wl_t, bl, wr_t)


def _sc_gather_pairs(h, edge_index, e0, EH):
    N, D = h.shape
    EW = EH // NW
    C = 40
    NB = 5
    n_waves = EW // (C * NB)

    dt = h.dtype
    mesh = plsc.VectorSubcoreMesh(core_axis_name="c", subcore_axis_name="s", num_cores=NC, num_subcores=NS)

    @functools.partial(
        pl.kernel,
        out_type=(jax.ShapeDtypeStruct((EH, D), dt),
                  jax.ShapeDtypeStruct((EH, D), dt)),
        mesh=mesh,
        scratch_types=[
            pltpu.VMEM((NB, 2, C), jnp.int32),
            pltpu.VMEM((NB, C, D), dt),
            pltpu.VMEM((NB, C, D), dt),
            pltpu.SemaphoreType.DMA,
            pltpu.SemaphoreType.DMA,
            pltpu.SemaphoreType.DMA,
        ],
        compiler_params=pltpu.CompilerParams(use_tc_tiling_on_sc=False))
    def k(h_hbm, ei_hbm, gv_hbm, gu_hbm, idx_v, rv, ru, si, sg, so):
        cid = lax.axis_index("c")
        sid = lax.axis_index("s")
        wid = cid * NS + sid

        def wave(w, _):
            base0 = wid * EW + w * (C * NB)
            loads = []
            for b in range(NB):
                base = e0 + base0 + b * C
                loads.append(pltpu.async_copy(
                    ei_hbm.at[0, pl.ds(base, C)], idx_v.at[b, 0], si))
                loads.append(pltpu.async_copy(
                    ei_hbm.at[1, pl.ds(base, C)], idx_v.at[b, 1], si))
            for d in loads:
                d.wait()
            gathers = []
            for b in range(NB):
                gathers.append(pltpu.async_copy(
                    h_hbm.at[idx_v.at[b, 0]], rv.at[b], sg))
                gathers.append(pltpu.async_copy(
                    h_hbm.at[idx_v.at[b, 1]], ru.at[b], sg))
            for d in gathers:
                d.wait()
            stores = []
            for b in range(NB):
                base = base0 + b * C
                stores.append(pltpu.async_copy(
                    rv.at[b], gv_hbm.at[pl.ds(base, C)], so))
                stores.append(pltpu.async_copy(
                    ru.at[b], gu_hbm.at[pl.ds(base, C)], so))
            for d in stores:
                d.wait()
            return 0
        lax.fori_loop(0, n_waves, wave, 0)

    return k(h, edge_index)


def _tc_edge_mlp(gv, gu, ef, w1v_t, w1u_t, w1e_t, b1,
                 w2_t, b2, w3_t, b3, w4_t, b4):
    E, D = gv.shape
    DE = ef.shape[1]
    H1 = w1v_t.shape[1]
    H2 = w2_t.shape[1]
    H3 = w3_t.shape[1]
    OUT = w4_t.shape[1]
    BE = 2000
    grid = (E // BE,)

    bf = jnp.bfloat16

    def body(gv_ref, gu_ref, ef_ref, w1v_ref, w1u_ref, w1e_ref, b1_ref,
             w2_ref, b2_ref, w3_ref, b3_ref, w4_ref, b4_ref, out_ref):
        h = jnp.dot(gv_ref[...], w1v_ref[...],
                    preferred_element_type=jnp.float32)
        h = h + jnp.dot(gu_ref[...], w1u_ref[...],
                        preferred_element_type=jnp.float32)
        h = h + jnp.dot(ef_ref[...], w1e_ref[...],
                        preferred_element_type=jnp.float32)
        h = jnp.maximum(h + b1_ref[...], 0.0)
        h = jnp.maximum(jnp.dot(h, w2_ref[...],
                                preferred_element_type=jnp.float32)
                        + b2_ref[...], 0.0)
        h = jnp.maximum(jnp.dot(h, w3_ref[...],
                                preferred_element_type=jnp.float32)
                        + b3_ref[...], 0.0)
        out_ref[...] = jnp.dot(h, w4_ref[...],
                               preferred_element_type=jnp.float32) + b4_ref[...]

    return pl.pallas_call(
        body,
        grid=grid,
        in_specs=[
            pl.BlockSpec((BE, D), lambda i: (i, 0)),
            pl.BlockSpec((BE, D), lambda i: (i, 0)),
            pl.BlockSpec((BE, DE), lambda i: (i, 0)),
            pl.BlockSpec((D, H1), lambda i: (0, 0)),
            pl.BlockSpec((D, H1), lambda i: (0, 0)),
            pl.BlockSpec((DE, H1), lambda i: (0, 0)),
            pl.BlockSpec((1, H1), lambda i: (0, 0)),
            pl.BlockSpec((H1, H2), lambda i: (0, 0)),
            pl.BlockSpec((1, H2), lambda i: (0, 0)),
            pl.BlockSpec((H2, H3), lambda i: (0, 0)),
            pl.BlockSpec((1, H3), lambda i: (0, 0)),
            pl.BlockSpec((H3, OUT), lambda i: (0, 0)),
            pl.BlockSpec((1, OUT), lambda i: (0, 0)),
        ],
        out_specs=pl.BlockSpec((BE, OUT), lambda i: (i, 0)),
        out_shape=jax.ShapeDtypeStruct((E, OUT), jnp.float32),
    )(gv, gu, ef, w1v_t, w1u_t, w1e_t, b1, w2_t, b2, w3_t, b3, w4_t, b4)


def kernel(x, edge_index, edge_features, num_nodes,
           W_l, b_l, W_r, W1, b1, W2, b2, W3, b3, W4, b4):
    del num_nodes
    D = x.shape[1]

    wl_t = W_l.T
    wr_t = W_r.T
    bl = b_l.reshape(1, -1)

    agg1, cntp = _sc_aggregate(x, edge_index, with_count=True)
    h1 = _tc_node_update(agg1, cntp, x, wl_t, bl, wr_t)
    agg2 = _sc_aggregate(h1, edge_index)
    h2 = _tc_node_update(agg2, cntp, h1, wl_t, bl, wr_t)

    E = edge_index.shape[1]
    EH = E // 2
    mlp_w = (W1[:, :D].T, W1[:, D:2 * D].T, W1[:, 2 * D:].T,
             b1.reshape(1, -1), W2.T, b2.reshape(1, -1),
             W3.T, b3.reshape(1, -1), W4.T, b4.reshape(1, -1))
    gv0, gu0 = _sc_gather_pairs(h2, edge_index, 0, EH)
    gv1, gu1 = _sc_gather_pairs(h2, edge_index, EH, E - EH)
    pred0 = _tc_edge_mlp(gv0, gu0, edge_features[:EH], *mlp_w)
    pred1 = _tc_edge_mlp(gv1, gu1, edge_features[EH:], *mlp_w)
    return jnp.concatenate([pred0, pred1], axis=0)


# speedup vs baseline: 1.6290x; 1.0181x over previous
import functools

import jax
import jax.numpy as jnp
from jax import lax
from jax.experimental import pallas as pl
from jax.experimental.pallas import tpu as pltpu
from jax.experimental.pallas import tpu_sc as plsc

NC = 2
NS = 16
NW = NC * NS
LANES = 16


def _node_padding(N):
    RPT = (-(-N // NS) + 7) // 8 * 8
    while RPT % 5:
        RPT += 8
    return RPT, RPT * NS


def _sc_aggregate(x, edge_index, with_count=False):
    N, D = x.shape
    E = edge_index.shape[1]
    EW = E // NW
    C = 40
    NB = 5
    n_waves = EW // (C * NB)
    RPT, NP = _node_padding(N)
    ZR = RPT // 10

    out_type = [jax.ShapeDtypeStruct((NC, NP, D), jnp.float32)]
    scratch = [
        pltpu.VMEM((NB, 2, C), jnp.int32),
        pltpu.VMEM((NB, C, D), jnp.float32),
        pltpu.VMEM((ZR, D), jnp.float32),
        pltpu.VMEM_SHARED((NP, D), jnp.float32),
        pltpu.SemaphoreType.DMA,
        pltpu.SemaphoreType.DMA,
        pltpu.SemaphoreType.DMA,
    ]
    if with_count:
        out_type.append(jax.ShapeDtypeStruct((NC, NP, LANES), jnp.float32))
        scratch += [
            pltpu.VMEM((C, LANES), jnp.float32),
            pltpu.VMEM((ZR, LANES), jnp.float32),
            pltpu.VMEM_SHARED((NP, LANES), jnp.float32),
        ]

    mesh = plsc.VectorSubcoreMesh(core_axis_name="c", subcore_axis_name="s", num_cores=NC, num_subcores=NS)

    def body(x_hbm, ei_hbm, *refs):
        if with_count:
            (agg_hbm, cnt_hbm, idx_v, rows_v, zero_v, agg_sp, si, sg, ss,
             ones_v, zcnt_v, cnt_sp) = refs
        else:
            (agg_hbm, idx_v, rows_v, zero_v, agg_sp, si, sg, ss) = refs
        cid = lax.axis_index("c")
        sid = lax.axis_index("s")
        wid = cid * NS + sid

        def zrow(i, _):
            for k in range(D // LANES):
                zero_v[i, pl.ds(LANES * k, LANES)] = jnp.zeros(
                    (LANES,), jnp.float32)
            return 0
        lax.fori_loop(0, ZR, zrow, 0)
        for j in range(RPT // ZR):
            pltpu.sync_copy(zero_v, agg_sp.at[pl.ds(sid * RPT + j * ZR, ZR)])
        if with_count:
            def orow(i, _):
                ones_v[i, pl.ds(0, LANES)] = jnp.ones((LANES,), jnp.float32)
                return 0
            lax.fori_loop(0, C, orow, 0)

            def zcrow(i, _):
                zcnt_v[i, pl.ds(0, LANES)] = jnp.zeros((LANES,), jnp.float32)
                return 0
            lax.fori_loop(0, ZR, zcrow, 0)
            for j in range(RPT // ZR):
                pltpu.sync_copy(
                    zcnt_v, cnt_sp.at[pl.ds(sid * RPT + j * ZR, ZR)])
        plsc.subcore_barrier()

        def wave(w, _):
            base0 = wid * EW + w * (C * NB)
            loads = []
            for b in range(NB):
                base = base0 + b * C
                loads.append(pltpu.async_copy(
                    ei_hbm.at[0, pl.ds(base, C)], idx_v.at[b, 0], si))
                loads.append(pltpu.async_copy(
                    ei_hbm.at[1, pl.ds(base, C)], idx_v.at[b, 1], si))
            for d in loads:
                d.wait()
            gathers = [pltpu.async_copy(x_hbm.at[idx_v.at[b, 0]],
                                        rows_v.at[b], sg)
                       for b in range(NB)]
            for d in gathers:
                d.wait()
            scatters = []
            for b in range(NB):
                scatters.append(pltpu.async_copy(
                    rows_v.at[b], agg_sp.at[idx_v.at[b, 1]], ss, add=True))
                if with_count:
                    scatters.append(pltpu.async_copy(
                        ones_v, cnt_sp.at[idx_v.at[b, 1]], ss, add=True))
            for d in scatters:
                d.wait()
            return 0
        lax.fori_loop(0, n_waves, wave, 0)

        plsc.subcore_barrier()
        pltpu.sync_copy(agg_sp.at[pl.ds(sid * RPT, RPT)],
                        agg_hbm.at[cid, pl.ds(sid * RPT, RPT)])
        if with_count:
            pltpu.sync_copy(cnt_sp.at[pl.ds(sid * RPT, RPT)],
                            cnt_hbm.at[cid, pl.ds(sid * RPT, RPT)])

    f = pl.kernel(body, out_type=tuple(out_type), mesh=mesh,
                  scratch_types=scratch,
                  compiler_params=pltpu.CompilerParams(
                      use_tc_tiling_on_sc=False))
    out = f(x, edge_index)
    return out if with_count else out[0]


def _tc_node_update(aggp, cntp, x, wl_t, bl, wr_t, out_dtype=jnp.float32):
    N, D = x.shape
    BN = 2000
    grid = (N // BN,)

    def body(agg_ref, cnt_ref, x_ref, wl_ref, bl_ref, wr_ref, out_ref):
        agg = agg_ref[0] + agg_ref[1]
        cnt = cnt_ref[0, :, 0:1] + cnt_ref[1, :, 0:1]
        mean = agg / jnp.maximum(cnt, 1.0)
        h = jnp.dot(mean, wl_ref[...], preferred_element_type=jnp.float32)
        h = h + jnp.dot(x_ref[...], wr_ref[...],
                        preferred_element_type=jnp.float32)
        out_ref[...] = jnp.maximum(h + bl_ref[...], 0.0).astype(out_dtype)

    return pl.pallas_call(
        body,
        grid=grid,
        in_specs=[
            pl.BlockSpec((NC, BN, D), lambda i: (0, i, 0)),
            pl.BlockSpec((NC, BN, LANES), lambda i: (0, i, 0)),
            pl.BlockSpec((BN, D), lambda i: (i, 0)),
            pl.BlockSpec((D, D), lambda i: (0, 0)),
            pl.BlockSpec((1, D), lambda i: (0, 0)),
            pl.BlockSpec((D, D), lambda i: (0, 0)),
        ],
        out_specs=pl.BlockSpec((BN, D), lambda i: (i, 0)),
        out_shape=jax.ShapeDtypeStruct((N, D), out_dtype),
    )(aggp, cntp, x, wl_t, bl, wr_t)


def _sc_gather_pairs(h, edge_index, e0, EH):
    N, D = h.shape
    EW = EH // NW
    C = 80
    NB = 5
    n_waves = EW // (C * NB)

    dt = h.dtype
    mesh = plsc.VectorSubcoreMesh(core_axis_name="c", subcore_axis_name="s", num_cores=NC, num_subcores=NS)

    @functools.partial(
        pl.kernel,
        out_type=(jax.ShapeDtypeStruct((EH, D), dt),
                  jax.ShapeDtypeStruct((EH, D), dt)),
        mesh=mesh,
        scratch_types=[
            pltpu.VMEM((NB, 2, C), jnp.int32),
            pltpu.VMEM((NB, C, D), dt),
            pltpu.VMEM((NB, C, D), dt),
            pltpu.SemaphoreType.DMA,
            pltpu.SemaphoreType.DMA,
            pltpu.SemaphoreType.DMA,
        ],
        compiler_params=pltpu.CompilerParams(use_tc_tiling_on_sc=False))
    def k(h_hbm, ei_hbm, gv_hbm, gu_hbm, idx_v, rv, ru, si, sg, so):
        cid = lax.axis_index("c")
        sid = lax.axis_index("s")
        wid = cid * NS + sid

        def wave(w, _):
            base0 = wid * EW + w * (C * NB)
            loads = []
            for b in range(NB):
                base = e0 + base0 + b * C
                loads.append(pltpu.async_copy(
                    ei_hbm.at[0, pl.ds(base, C)], idx_v.at[b, 0], si))
                loads.append(pltpu.async_copy(
                    ei_hbm.at[1, pl.ds(base, C)], idx_v.at[b, 1], si))
            for d in loads:
                d.wait()
            gathers = []
            for b in range(NB):
                gathers.append(pltpu.async_copy(
                    h_hbm.at[idx_v.at[b, 0]], rv.at[b], sg))
                gathers.append(pltpu.async_copy(
                    h_hbm.at[idx_v.at[b, 1]], ru.at[b], sg))
            for d in gathers:
                d.wait()
            stores = []
            for b in range(NB):
                base = base0 + b * C
                stores.append(pltpu.async_copy(
                    rv.at[b], gv_hbm.at[pl.ds(base, C)], so))
                stores.append(pltpu.async_copy(
                    ru.at[b], gu_hbm.at[pl.ds(base, C)], so))
            for d in stores:
                d.wait()
            return 0
        lax.fori_loop(0, n_waves, wave, 0)

    return k(h, edge_index)


def _tc_edge_mlp(gv, gu, ef, ef_off, w1v_t, w1u_t, w1e_t, b1,
                 w2_t, b2, w3_t, b3, w4_t, b4):
    E, D = gv.shape
    DE = ef.shape[1]
    H1 = w1v_t.shape[1]
    H2 = w2_t.shape[1]
    H3 = w3_t.shape[1]
    OUT = w4_t.shape[1]
    BE = 2000
    grid = (E // BE,)

    bf = jnp.bfloat16

    def body(gv_ref, gu_ref, ef_ref, w1v_ref, w1u_ref, w1e_ref, b1_ref,
             w2_ref, b2_ref, w3_ref, b3_ref, w4_ref, b4_ref, out_ref):
        h = jnp.dot(gv_ref[...], w1v_ref[...],
                    preferred_element_type=jnp.float32)
        h = h + jnp.dot(gu_ref[...], w1u_ref[...],
                        preferred_element_type=jnp.float32)
        h = h + jnp.dot(ef_ref[...], w1e_ref[...],
                        preferred_element_type=jnp.float32)
        h = jnp.maximum(h + b1_ref[...], 0.0)
        h = jnp.maximum(jnp.dot(h, w2_ref[...],
                                preferred_element_type=jnp.float32)
                        + b2_ref[...], 0.0)
        h = jnp.maximum(jnp.dot(h, w3_ref[...],
                                preferred_element_type=jnp.float32)
                        + b3_ref[...], 0.0)
        out_ref[...] = jnp.dot(h, w4_ref[...],
                               preferred_element_type=jnp.float32) + b4_ref[...]

    return pl.pallas_call(
        body,
        grid=grid,
        in_specs=[
            pl.BlockSpec((BE, D), lambda i: (i, 0)),
            pl.BlockSpec((BE, D), lambda i: (i, 0)),
            pl.BlockSpec((BE, DE), lambda i: (i + ef_off // BE, 0)),
            pl.BlockSpec((D, H1), lambda i: (0, 0)),
            pl.BlockSpec((D, H1), lambda i: (0, 0)),
            pl.BlockSpec((DE, H1), lambda i: (0, 0)),
            pl.BlockSpec((1, H1), lambda i: (0, 0)),
            pl.BlockSpec((H1, H2), lambda i: (0, 0)),
            pl.BlockSpec((1, H2), lambda i: (0, 0)),
            pl.BlockSpec((H2, H3), lambda i: (0, 0)),
            pl.BlockSpec((1, H3), lambda i: (0, 0)),
            pl.BlockSpec((H3, OUT), lambda i: (0, 0)),
            pl.BlockSpec((1, OUT), lambda i: (0, 0)),
        ],
        out_specs=pl.BlockSpec((BE, OUT), lambda i: (i, 0)),
        out_shape=jax.ShapeDtypeStruct((E, OUT), jnp.float32),
    )(gv, gu, ef, w1v_t, w1u_t, w1e_t, b1, w2_t, b2, w3_t, b3, w4_t, b4)


def kernel(x, edge_index, edge_features, num_nodes,
           W_l, b_l, W_r, W1, b1, W2, b2, W3, b3, W4, b4):
    del num_nodes
    D = x.shape[1]

    wl_t = W_l.T
    wr_t = W_r.T
    bl = b_l.reshape(1, -1)

    agg1, cntp = _sc_aggregate(x, edge_index, with_count=True)
    h1 = _tc_node_update(agg1, cntp, x, wl_t, bl, wr_t)
    agg2 = _sc_aggregate(h1, edge_index)
    h2 = _tc_node_update(agg2, cntp, h1, wl_t, bl, wr_t)

    E = edge_index.shape[1]
    EH = E * 3 // 5
    mlp_w = (W1[:, :D].T, W1[:, D:2 * D].T, W1[:, 2 * D:].T,
             b1.reshape(1, -1), W2.T, b2.reshape(1, -1),
             W3.T, b3.reshape(1, -1), W4.T, b4.reshape(1, -1))
    gv0, gu0 = _sc_gather_pairs(h2, edge_index, 0, EH)
    gv1, gu1 = _sc_gather_pairs(h2, edge_index, EH, E - EH)
    pred0 = _tc_edge_mlp(gv0, gu0, edge_features, 0, *mlp_w)
    pred1 = _tc_edge_mlp(gv1, gu1, edge_features, EH, *mlp_w)
    return jnp.concatenate([pred0, pred1], axis=0)


# speedup vs baseline: 1.6443x; 1.0094x over previous
import functools

import jax
import jax.numpy as jnp
from jax import lax
from jax.experimental import pallas as pl
from jax.experimental.pallas import tpu as pltpu
from jax.experimental.pallas import tpu_sc as plsc

NC = 2
NS = 16
NW = NC * NS
LANES = 16


def _node_padding(N):
    RPT = (-(-N // NS) + 7) // 8 * 8
    while RPT % 5:
        RPT += 8
    return RPT, RPT * NS


def _sc_aggregate(x, edge_index, with_count=False):
    N, D = x.shape
    E = edge_index.shape[1]
    EW = E // NW
    C = 40
    NB = 5
    n_waves = EW // (C * NB)
    RPT, NP = _node_padding(N)
    ZR = RPT // 10

    out_type = [jax.ShapeDtypeStruct((NC, NP, D), jnp.float32)]
    scratch = [
        pltpu.VMEM((NB, 2, C), jnp.int32),
        pltpu.VMEM((NB, C, D), jnp.float32),
        pltpu.VMEM((ZR, D), jnp.float32),
        pltpu.VMEM_SHARED((NP, D), jnp.float32),
        pltpu.SemaphoreType.DMA,
        [pltpu.SemaphoreType.DMA] * NB,
        pltpu.SemaphoreType.DMA,
    ]
    if with_count:
        out_type.append(jax.ShapeDtypeStruct((NC, NP, LANES), jnp.float32))
        scratch += [
            pltpu.VMEM((C, LANES), jnp.float32),
            pltpu.VMEM((ZR, LANES), jnp.float32),
            pltpu.VMEM_SHARED((NP, LANES), jnp.float32),
        ]

    mesh = plsc.VectorSubcoreMesh(core_axis_name="c", subcore_axis_name="s", num_cores=NC, num_subcores=NS)

    def body(x_hbm, ei_hbm, *refs):
        if with_count:
            (agg_hbm, cnt_hbm, idx_v, rows_v, zero_v, agg_sp, si, sg, ss,
             ones_v, zcnt_v, cnt_sp) = refs
        else:
            (agg_hbm, idx_v, rows_v, zero_v, agg_sp, si, sg, ss) = refs
        cid = lax.axis_index("c")
        sid = lax.axis_index("s")
        wid = cid * NS + sid

        def zrow(i, _):
            for k in range(D // LANES):
                zero_v[i, pl.ds(LANES * k, LANES)] = jnp.zeros(
                    (LANES,), jnp.float32)
            return 0
        lax.fori_loop(0, ZR, zrow, 0)
        for j in range(RPT // ZR):
            pltpu.sync_copy(zero_v, agg_sp.at[pl.ds(sid * RPT + j * ZR, ZR)])
        if with_count:
            def orow(i, _):
                ones_v[i, pl.ds(0, LANES)] = jnp.ones((LANES,), jnp.float32)
                return 0
            lax.fori_loop(0, C, orow, 0)

            def zcrow(i, _):
                zcnt_v[i, pl.ds(0, LANES)] = jnp.zeros((LANES,), jnp.float32)
                return 0
            lax.fori_loop(0, ZR, zcrow, 0)
            for j in range(RPT // ZR):
                pltpu.sync_copy(
                    zcnt_v, cnt_sp.at[pl.ds(sid * RPT + j * ZR, ZR)])
        plsc.subcore_barrier()

        def wave(w, _):
            base0 = wid * EW + w * (C * NB)
            loads = []
            for b in range(NB):
                base = base0 + b * C
                loads.append(pltpu.async_copy(
                    ei_hbm.at[0, pl.ds(base, C)], idx_v.at[b, 0], si))
                loads.append(pltpu.async_copy(
                    ei_hbm.at[1, pl.ds(base, C)], idx_v.at[b, 1], si))
            for d in loads:
                d.wait()
            gathers = [pltpu.async_copy(x_hbm.at[idx_v.at[b, 0]],
                                        rows_v.at[b], sg[b])
                       for b in range(NB)]
            scatters = []
            for b in range(NB):
                gathers[b].wait()
                scatters.append(pltpu.async_copy(
                    rows_v.at[b], agg_sp.at[idx_v.at[b, 1]], ss, add=True))
                if with_count:
                    scatters.append(pltpu.async_copy(
                        ones_v, cnt_sp.at[idx_v.at[b, 1]], ss, add=True))
            for d in scatters:
                d.wait()
            return 0
        lax.fori_loop(0, n_waves, wave, 0)

        plsc.subcore_barrier()
        pltpu.sync_copy(agg_sp.at[pl.ds(sid * RPT, RPT)],
                        agg_hbm.at[cid, pl.ds(sid * RPT, RPT)])
        if with_count:
            pltpu.sync_copy(cnt_sp.at[pl.ds(sid * RPT, RPT)],
                            cnt_hbm.at[cid, pl.ds(sid * RPT, RPT)])

    f = pl.kernel(body, out_type=tuple(out_type), mesh=mesh,
                  scratch_types=scratch,
                  compiler_params=pltpu.CompilerParams(
                      use_tc_tiling_on_sc=False))
    out = f(x, edge_index)
    return out if with_count else out[0]


def _tc_node_update(aggp, cntp, x, wl_t, bl, wr_t, out_dtype=jnp.float32):
    N, D = x.shape
    BN = 2000
    grid = (N // BN,)

    def body(agg_ref, cnt_ref, x_ref, wl_ref, bl_ref, wr_ref, out_ref):
        agg = agg_ref[0] + agg_ref[1]
        cnt = cnt_ref[0, :, 0:1] + cnt_ref[1, :, 0:1]
        mean = agg / jnp.maximum(cnt, 1.0)
        h = jnp.dot(mean, wl_ref[...], preferred_element_type=jnp.float32)
        h = h + jnp.dot(x_ref[...], wr_ref[...],
                        preferred_element_type=jnp.float32)
        out_ref[...] = jnp.maximum(h + bl_ref[...], 0.0).astype(out_dtype)

    return pl.pallas_call(
        body,
        grid=grid,
        in_specs=[
            pl.BlockSpec((NC, BN, D), lambda i: (0, i, 0)),
            pl.BlockSpec((NC, BN, LANES), lambda i: (0, i, 0)),
            pl.BlockSpec((BN, D), lambda i: (i, 0)),
            pl.BlockSpec((D, D), lambda i: (0, 0)),
            pl.BlockSpec((1, D), lambda i: (0, 0)),
            pl.BlockSpec((D, D), lambda i: (0, 0)),
        ],
        out_specs=pl.BlockSpec((BN, D), lambda i: (i, 0)),
        out_shape=jax.ShapeDtypeStruct((N, D), out_dtype),
    )(aggp, cntp, x, wl_t, bl, wr_t)


def _sc_gather_pairs(h, edge_index, e0, EH):
    N, D = h.shape
    EW = EH // NW
    C = 80
    NB = 5
    n_waves = EW // (C * NB)

    dt = h.dtype
    mesh = plsc.VectorSubcoreMesh(core_axis_name="c", subcore_axis_name="s", num_cores=NC, num_subcores=NS)

    @functools.partial(
        pl.kernel,
        out_type=(jax.ShapeDtypeStruct((EH, D), dt),
                  jax.ShapeDtypeStruct((EH, D), dt)),
        mesh=mesh,
        scratch_types=[
            pltpu.VMEM((NB, 2, C), jnp.int32),
            pltpu.VMEM((NB, C, D), dt),
            pltpu.VMEM((NB, C, D), dt),
            pltpu.SemaphoreType.DMA,
            [pltpu.SemaphoreType.DMA] * NB,
            pltpu.SemaphoreType.DMA,
        ],
        compiler_params=pltpu.CompilerParams(use_tc_tiling_on_sc=False))
    def k(h_hbm, ei_hbm, gv_hbm, gu_hbm, idx_v, rv, ru, si, sg, so):
        cid = lax.axis_index("c")
        sid = lax.axis_index("s")
        wid = cid * NS + sid

        def wave(w, _):
            base0 = wid * EW + w * (C * NB)
            loads = []
            for b in range(NB):
                base = e0 + base0 + b * C
                loads.append(pltpu.async_copy(
                    ei_hbm.at[0, pl.ds(base, C)], idx_v.at[b, 0], si))
                loads.append(pltpu.async_copy(
                    ei_hbm.at[1, pl.ds(base, C)], idx_v.at[b, 1], si))
            for d in loads:
                d.wait()
            gathers = []
            for b in range(NB):
                gathers.append(pltpu.async_copy(
                    h_hbm.at[idx_v.at[b, 0]], rv.at[b], sg[b]))
                gathers.append(pltpu.async_copy(
                    h_hbm.at[idx_v.at[b, 1]], ru.at[b], sg[b]))
            stores = []
            for b in range(NB):
                gathers[2 * b].wait()
                gathers[2 * b + 1].wait()
                base = base0 + b * C
                stores.append(pltpu.async_copy(
                    rv.at[b], gv_hbm.at[pl.ds(base, C)], so))
                stores.append(pltpu.async_copy(
                    ru.at[b], gu_hbm.at[pl.ds(base, C)], so))
            for d in stores:
                d.wait()
            return 0
        lax.fori_loop(0, n_waves, wave, 0)

    return k(h, edge_index)


def _tc_edge_mlp(gv, gu, ef, ef_off, w1v_t, w1u_t, w1e_t, b1,
                 w2_t, b2, w3_t, b3, w4_t, b4):
    E, D = gv.shape
    DE = ef.shape[1]
    H1 = w1v_t.shape[1]
    H2 = w2_t.shape[1]
    H3 = w3_t.shape[1]
    OUT = w4_t.shape[1]
    BE = 2000
    grid = (E // BE,)

    bf = jnp.bfloat16

    def body(gv_ref, gu_ref, ef_ref, w1v_ref, w1u_ref, w1e_ref, b1_ref,
             w2_ref, b2_ref, w3_ref, b3_ref, w4_ref, b4_ref, out_ref):
        h = jnp.dot(gv_ref[...], w1v_ref[...],
                    preferred_element_type=jnp.float32)
        h = h + jnp.dot(gu_ref[...], w1u_ref[...],
                        preferred_element_type=jnp.float32)
        h = h + jnp.dot(ef_ref[...], w1e_ref[...],
                        preferred_element_type=jnp.float32)
        h = jnp.maximum(h + b1_ref[...], 0.0)
        h = jnp.maximum(jnp.dot(h, w2_ref[...],
                                preferred_element_type=jnp.float32)
                        + b2_ref[...], 0.0)
        h = jnp.maximum(jnp.dot(h, w3_ref[...],
                                preferred_element_type=jnp.float32)
                        + b3_ref[...], 0.0)
        out_ref[...] = jnp.dot(h, w4_ref[...],
                               preferred_element_type=jnp.float32) + b4_ref[...]

    return pl.pallas_call(
        body,
        grid=grid,
        in_specs=[
            pl.BlockSpec((BE, D), lambda i: (i, 0)),
            pl.BlockSpec((BE, D), lambda i: (i, 0)),
            pl.BlockSpec((BE, DE), lambda i: (i + ef_off // BE, 0)),
            pl.BlockSpec((D, H1), lambda i: (0, 0)),
            pl.BlockSpec((D, H1), lambda i: (0, 0)),
            pl.BlockSpec((DE, H1), lambda i: (0, 0)),
            pl.BlockSpec((1, H1), lambda i: (0, 0)),
            pl.BlockSpec((H1, H2), lambda i: (0, 0)),
            pl.BlockSpec((1, H2), lambda i: (0, 0)),
            pl.BlockSpec((H2, H3), lambda i: (0, 0)),
            pl.BlockSpec((1, H3), lambda i: (0, 0)),
            pl.BlockSpec((H3, OUT), lambda i: (0, 0)),
            pl.BlockSpec((1, OUT), lambda i: (0, 0)),
        ],
        out_specs=pl.BlockSpec((BE, OUT), lambda i: (i, 0)),
        out_shape=jax.ShapeDtypeStruct((E, OUT), jnp.float32),
    )(gv, gu, ef, w1v_t, w1u_t, w1e_t, b1, w2_t, b2, w3_t, b3, w4_t, b4)


def kernel(x, edge_index, edge_features, num_nodes,
           W_l, b_l, W_r, W1, b1, W2, b2, W3, b3, W4, b4):
    del num_nodes
    D = x.shape[1]

    wl_t = W_l.T
    wr_t = W_r.T
    bl = b_l.reshape(1, -1)

    agg1, cntp = _sc_aggregate(x, edge_index, with_count=True)
    h1 = _tc_node_update(agg1, cntp, x, wl_t, bl, wr_t)
    agg2 = _sc_aggregate(h1, edge_index)
    h2 = _tc_node_update(agg2, cntp, h1, wl_t, bl, wr_t)

    E = edge_index.shape[1]
    EH = E * 3 // 5
    mlp_w = (W1[:, :D].T, W1[:, D:2 * D].T, W1[:, 2 * D:].T,
             b1.reshape(1, -1), W2.T, b2.reshape(1, -1),
             W3.T, b3.reshape(1, -1), W4.T, b4.reshape(1, -1))
    gv0, gu0 = _sc_gather_pairs(h2, edge_index, 0, EH)
    gv1, gu1 = _sc_gather_pairs(h2, edge_index, EH, E - EH)
    pred0 = _tc_edge_mlp(gv0, gu0, edge_features, 0, *mlp_w)
    pred1 = _tc_edge_mlp(gv1, gu1, edge_features, EH, *mlp_w)
    return jnp.concatenate([pred0, pred1], axis=0)


# speedup vs baseline: 1.6596x; 1.0093x over previous
import functools

import jax
import jax.numpy as jnp
from jax import lax
from jax.experimental import pallas as pl
from jax.experimental.pallas import tpu as pltpu
from jax.experimental.pallas import tpu_sc as plsc

NC = 2
NS = 16
NW = NC * NS
LANES = 16


def _node_padding(N):
    RPT = (-(-N // NS) + 7) // 8 * 8
    while RPT % 5:
        RPT += 8
    return RPT, RPT * NS


def _sc_aggregate(x, edge_index, with_count=False):
    N, D = x.shape
    E = edge_index.shape[1]
    EW = E // NW
    C = 40
    NB = 5
    n_waves = EW // (C * NB)
    RPT, NP = _node_padding(N)
    ZR = RPT // 10

    out_type = [jax.ShapeDtypeStruct((NC, NP, D), jnp.float32)]
    scratch = [
        pltpu.VMEM((NB, 2, C), jnp.int32),
        pltpu.VMEM((NB, C, D), jnp.float32),
        pltpu.VMEM((ZR, D), jnp.float32),
        pltpu.VMEM_SHARED((NP, D), jnp.float32),
        pltpu.SemaphoreType.DMA,
        [pltpu.SemaphoreType.DMA] * NB,
        pltpu.SemaphoreType.DMA,
    ]
    if with_count:
        out_type.append(jax.ShapeDtypeStruct((NC, NP, LANES), jnp.float32))
        scratch += [
            pltpu.VMEM((C, LANES), jnp.float32),
            pltpu.VMEM((ZR, LANES), jnp.float32),
            pltpu.VMEM_SHARED((NP, LANES), jnp.float32),
        ]

    mesh = plsc.VectorSubcoreMesh(core_axis_name="c", subcore_axis_name="s", num_cores=NC, num_subcores=NS)

    def body(x_hbm, ei_hbm, *refs):
        if with_count:
            (agg_hbm, cnt_hbm, idx_v, rows_v, zero_v, agg_sp, si, sg, ss,
             ones_v, zcnt_v, cnt_sp) = refs
        else:
            (agg_hbm, idx_v, rows_v, zero_v, agg_sp, si, sg, ss) = refs
        cid = lax.axis_index("c")
        sid = lax.axis_index("s")
        wid = cid * NS + sid

        def zrow(i, _):
            for k in range(D // LANES):
                zero_v[i, pl.ds(LANES * k, LANES)] = jnp.zeros(
                    (LANES,), jnp.float32)
            return 0
        lax.fori_loop(0, ZR, zrow, 0)
        for j in range(RPT // ZR):
            pltpu.sync_copy(zero_v, agg_sp.at[pl.ds(sid * RPT + j * ZR, ZR)])
        if with_count:
            def orow(i, _):
                ones_v[i, pl.ds(0, LANES)] = jnp.ones((LANES,), jnp.float32)
                return 0
            lax.fori_loop(0, C, orow, 0)

            def zcrow(i, _):
                zcnt_v[i, pl.ds(0, LANES)] = jnp.zeros((LANES,), jnp.float32)
                return 0
            lax.fori_loop(0, ZR, zcrow, 0)
            for j in range(RPT // ZR):
                pltpu.sync_copy(
                    zcnt_v, cnt_sp.at[pl.ds(sid * RPT + j * ZR, ZR)])
        plsc.subcore_barrier()

        def wave(w, _):
            base0 = wid * EW + w * (C * NB)
            loads = []
            for b in range(NB):
                base = base0 + b * C
                loads.append(pltpu.async_copy(
                    ei_hbm.at[0, pl.ds(base, C)], idx_v.at[b, 0], si))
                loads.append(pltpu.async_copy(
                    ei_hbm.at[1, pl.ds(base, C)], idx_v.at[b, 1], si))
            for d in loads:
                d.wait()
            gathers = [pltpu.async_copy(x_hbm.at[idx_v.at[b, 0]],
                                        rows_v.at[b], sg[b])
                       for b in range(NB)]
            scatters = []
            for b in range(NB):
                gathers[b].wait()
                scatters.append(pltpu.async_copy(
                    rows_v.at[b], agg_sp.at[idx_v.at[b, 1]], ss, add=True))
                if with_count:
                    scatters.append(pltpu.async_copy(
                        ones_v, cnt_sp.at[idx_v.at[b, 1]], ss, add=True))
            for d in scatters:
                d.wait()
            return 0
        lax.fori_loop(0, n_waves, wave, 0)

        plsc.subcore_barrier()
        pltpu.sync_copy(agg_sp.at[pl.ds(sid * RPT, RPT)],
                        agg_hbm.at[cid, pl.ds(sid * RPT, RPT)])
        if with_count:
            pltpu.sync_copy(cnt_sp.at[pl.ds(sid * RPT, RPT)],
                            cnt_hbm.at[cid, pl.ds(sid * RPT, RPT)])

    f = pl.kernel(body, out_type=tuple(out_type), mesh=mesh,
                  scratch_types=scratch,
                  compiler_params=pltpu.CompilerParams(
                      use_tc_tiling_on_sc=False))
    out = f(x, edge_index)
    return out if with_count else out[0]


def _tc_node_update(aggp, cntp, x, wl_t, bl, wr_t, out_dtype=jnp.float32):
    N, D = x.shape
    BN = 2000
    grid = (N // BN,)

    def body(agg_ref, cnt_ref, x_ref, wl_ref, bl_ref, wr_ref, out_ref):
        agg = agg_ref[0] + agg_ref[1]
        cnt = cnt_ref[0, :, 0:1] + cnt_ref[1, :, 0:1]
        mean = agg / jnp.maximum(cnt, 1.0)
        h = jnp.dot(mean, wl_ref[...], preferred_element_type=jnp.float32)
        h = h + jnp.dot(x_ref[...], wr_ref[...],
                        preferred_element_type=jnp.float32)
        out_ref[...] = jnp.maximum(h + bl_ref[...], 0.0).astype(out_dtype)

    return pl.pallas_call(
        body,
        grid=grid,
        in_specs=[
            pl.BlockSpec((NC, BN, D), lambda i: (0, i, 0)),
            pl.BlockSpec((NC, BN, LANES), lambda i: (0, i, 0)),
            pl.BlockSpec((BN, D), lambda i: (i, 0)),
            pl.BlockSpec((D, D), lambda i: (0, 0)),
            pl.BlockSpec((1, D), lambda i: (0, 0)),
            pl.BlockSpec((D, D), lambda i: (0, 0)),
        ],
        out_specs=pl.BlockSpec((BN, D), lambda i: (i, 0)),
        out_shape=jax.ShapeDtypeStruct((N, D), out_dtype),
    )(aggp, cntp, x, wl_t, bl, wr_t)


def _sc_gather_pairs(h, edge_index, e0, EH):
    N, D = h.shape
    EW = EH // NW
    C = 80
    NB = 5
    n_waves = EW // (C * NB)

    dt = h.dtype
    mesh = plsc.VectorSubcoreMesh(core_axis_name="c", subcore_axis_name="s", num_cores=NC, num_subcores=NS)

    @functools.partial(
        pl.kernel,
        out_type=(jax.ShapeDtypeStruct((EH, D), dt),
                  jax.ShapeDtypeStruct((EH, D), dt)),
        mesh=mesh,
        scratch_types=[
            pltpu.VMEM((NB, 2, C), jnp.int32),
            pltpu.VMEM((NB, C, D), dt),
            pltpu.VMEM((NB, C, D), dt),
            pltpu.SemaphoreType.DMA,
            [pltpu.SemaphoreType.DMA] * NB,
            pltpu.SemaphoreType.DMA,
        ],
        compiler_params=pltpu.CompilerParams(use_tc_tiling_on_sc=False))
    def k(h_hbm, ei_hbm, gv_hbm, gu_hbm, idx_v, rv, ru, si, sg, so):
        cid = lax.axis_index("c")
        sid = lax.axis_index("s")
        wid = cid * NS + sid

        def wave(w, _):
            base0 = wid * EW + w * (C * NB)
            loads = []
            for b in range(NB):
                base = e0 + base0 + b * C
                loads.append(pltpu.async_copy(
                    ei_hbm.at[0, pl.ds(base, C)], idx_v.at[b, 0], si))
                loads.append(pltpu.async_copy(
                    ei_hbm.at[1, pl.ds(base, C)], idx_v.at[b, 1], si))
            for d in loads:
                d.wait()
            gathers = []
            for b in range(NB):
                gathers.append(pltpu.async_copy(
                    h_hbm.at[idx_v.at[b, 0]], rv.at[b], sg[b]))
                gathers.append(pltpu.async_copy(
                    h_hbm.at[idx_v.at[b, 1]], ru.at[b], sg[b]))
            stores = []
            for b in range(NB):
                gathers[2 * b].wait()
                gathers[2 * b + 1].wait()
                base = base0 + b * C
                stores.append(pltpu.async_copy(
                    rv.at[b], gv_hbm.at[pl.ds(base, C)], so))
                stores.append(pltpu.async_copy(
                    ru.at[b], gu_hbm.at[pl.ds(base, C)], so))
            for d in stores:
                d.wait()
            return 0
        lax.fori_loop(0, n_waves, wave, 0)

    return k(h, edge_index)


def _tc_edge_mlp(gv, gu, ef, ef_off, w1v_t, w1u_t, w1e_t, b1,
                 w2_t, b2, w3_t, b3, w4_t, b4):
    E, D = gv.shape
    DE = ef.shape[1]
    H1 = w1v_t.shape[1]
    H2 = w2_t.shape[1]
    H3 = w3_t.shape[1]
    OUT = w4_t.shape[1]
    BE = 4000
    grid = (E // BE,)

    bf = jnp.bfloat16

    def body(gv_ref, gu_ref, ef_ref, w1v_ref, w1u_ref, w1e_ref, b1_ref,
             w2_ref, b2_ref, w3_ref, b3_ref, w4_ref, b4_ref, out_ref):
        h = jnp.dot(gv_ref[...], w1v_ref[...],
                    preferred_element_type=jnp.float32)
        h = h + jnp.dot(gu_ref[...], w1u_ref[...],
                        preferred_element_type=jnp.float32)
        h = h + jnp.dot(ef_ref[...], w1e_ref[...],
                        preferred_element_type=jnp.float32)
        h = jnp.maximum(h + b1_ref[...], 0.0)
        h = jnp.maximum(jnp.dot(h, w2_ref[...],
                                preferred_element_type=jnp.float32)
                        + b2_ref[...], 0.0)
        h = jnp.maximum(jnp.dot(h, w3_ref[...],
                                preferred_element_type=jnp.float32)
                        + b3_ref[...], 0.0)
        out_ref[...] = jnp.dot(h, w4_ref[...],
                               preferred_element_type=jnp.float32) + b4_ref[...]

    return pl.pallas_call(
        body,
        grid=grid,
        in_specs=[
            pl.BlockSpec((BE, D), lambda i: (i, 0)),
            pl.BlockSpec((BE, D), lambda i: (i, 0)),
            pl.BlockSpec((BE, DE), lambda i: (i + ef_off // BE, 0)),
            pl.BlockSpec((D, H1), lambda i: (0, 0)),
            pl.BlockSpec((D, H1), lambda i: (0, 0)),
            pl.BlockSpec((DE, H1), lambda i: (0, 0)),
            pl.BlockSpec((1, H1), lambda i: (0, 0)),
            pl.BlockSpec((H1, H2), lambda i: (0, 0)),
            pl.BlockSpec((1, H2), lambda i: (0, 0)),
            pl.BlockSpec((H2, H3), lambda i: (0, 0)),
            pl.BlockSpec((1, H3), lambda i: (0, 0)),
            pl.BlockSpec((H3, OUT), lambda i: (0, 0)),
            pl.BlockSpec((1, OUT), lambda i: (0, 0)),
        ],
        out_specs=pl.BlockSpec((BE, OUT), lambda i: (i, 0)),
        out_shape=jax.ShapeDtypeStruct((E, OUT), jnp.float32),
    )(gv, gu, ef, w1v_t, w1u_t, w1e_t, b1, w2_t, b2, w3_t, b3, w4_t, b4)


def kernel(x, edge_index, edge_features, num_nodes,
           W_l, b_l, W_r, W1, b1, W2, b2, W3, b3, W4, b4):
    del num_nodes
    D = x.shape[1]

    wl_t = W_l.T
    wr_t = W_r.T
    bl = b_l.reshape(1, -1)

    agg1, cntp = _sc_aggregate(x, edge_index, with_count=True)
    h1 = _tc_node_update(agg1, cntp, x, wl_t, bl, wr_t)
    agg2 = _sc_aggregate(h1, edge_index)
    h2 = _tc_node_update(agg2, cntp, h1, wl_t, bl, wr_t)

    E = edge_index.shape[1]
    mlp_w = (W1[:, :D].T, W1[:, D:2 * D].T, W1[:, 2 * D:].T,
             b1.reshape(1, -1), W2.T, b2.reshape(1, -1),
             W3.T, b3.reshape(1, -1), W4.T, b4.reshape(1, -1))
    splits = [0, E * 2 // 5, E * 4 // 5, E]
    preds = []
    for lo, hi in zip(splits[:-1], splits[1:]):
        gv, gu = _sc_gather_pairs(h2, edge_index, lo, hi - lo)
        preds.append(_tc_edge_mlp(gv, gu, edge_features, lo, *mlp_w))
    return jnp.concatenate(preds, axis=0)
